# baseline (speedup 1.0000x reference)
import functools

import jax
import jax.numpy as jnp
from jax import lax
from jax.experimental import pallas as pl
from jax.experimental.pallas import tpu as pltpu
from jax.experimental.pallas import tpu_sc as plsc

_NC = 2
_NS = 16
_NW = _NC * _NS
_L = 16

_BATCH = 16384
_D = 32
_BPW = _BATCH // _NW
_CHUNK = 128
_NCH = _BPW // _CHUNK

_MUL_C1 = 32 * 32
_MUL_U0 = 32 + 1

_mesh = plsc.VectorSubcoreMesh(core_axis_name="c", subcore_axis_name="s")


@functools.partial(
    pl.kernel,
    mesh=_mesh,
    compiler_params=pltpu.CompilerParams(use_tc_tiling_on_sc=False),
    out_type=(
        jax.ShapeDtypeStruct((_BATCH, _D), jnp.float32),
        jax.ShapeDtypeStruct((_BATCH, _D), jnp.float32),
    ),
    scratch_types=[
        pltpu.VMEM((_NCH, _CHUNK), jnp.int32),
        pltpu.VMEM((_NCH, _CHUNK), jnp.int32),
        pltpu.VMEM((_NCH, _CHUNK), jnp.int32),
        pltpu.VMEM((_NCH, _CHUNK), jnp.int32),
        pltpu.VMEM((_BPW, _D), jnp.float32),
        pltpu.VMEM((_BPW, _D), jnp.float32),
        pltpu.SemaphoreType.DMA,
        pltpu.SemaphoreType.DMA,
        pltpu.SemaphoreType.DMA,
    ],
)
def _qnet_gather(c0_hbm, c1_hbm, u0_hbm, w0_hbm, w1_hbm, q0_hbm, q1_hbm,
                 idx0_v, c1_v, u0_v, joint_v, rows0_v, rows1_v,
                 sem0, sem1, sem2):
    wid = lax.axis_index("s") * _NC + lax.axis_index("c")
    base = wid * _BPW

    pltpu.sync_copy(c0_hbm.at[wid], idx0_v)

    copies = []
    for j in range(_NCH):
        copies.append(
            pltpu.async_copy(
                w0_hbm.at[idx0_v.at[jnp.int32(j)]],
                rows0_v.at[pl.ds(j * _CHUNK, _CHUNK)],
                sem0,
            )
        )

    pltpu.sync_copy(c1_hbm.at[wid], c1_v)
    pltpu.sync_copy(u0_hbm.at[wid], u0_v)

    for j in range(_NCH):
        j32 = jnp.int32(j)
        for k in range(_CHUNK // _L):
            sl = pl.ds(k * _L, _L)
            joint_v[j32, sl] = c1_v[j32, sl] * _MUL_C1 + u0_v[j32, sl] * _MUL_U0

    for j in range(_NCH):
        copies.append(
            pltpu.async_copy(
                w1_hbm.at[joint_v.at[jnp.int32(j)]],
                rows1_v.at[pl.ds(j * _CHUNK, _CHUNK)],
                sem1,
            )
        )

    for c in copies:
        c.wait()

    out0 = pltpu.async_copy(rows0_v, q0_hbm.at[pl.ds(base, _BPW)], sem2)
    out1 = pltpu.async_copy(rows1_v, q1_hbm.at[pl.ds(base, _BPW)], sem2)
    out0.wait()
    out1.wait()


def kernel(cards_0, cards_1, u0, weights_0, weights_1):
    c0 = cards_0.astype(jnp.int32).reshape(_NW, _NCH, _CHUNK)
    c1 = cards_1.astype(jnp.int32).reshape(_NW, _NCH, _CHUNK)
    u0i = u0.astype(jnp.int32).reshape(_NW, _NCH, _CHUNK)
    w0 = weights_0.astype(jnp.float32)
    w1 = weights_1.astype(jnp.float32)
    q0, q1 = _qnet_gather(c0, c1, u0i, w0, w1)
    return (q0, q1)


# speedup vs baseline: 8.4195x; 8.4195x over previous
import functools

import jax
import jax.numpy as jnp
from jax import lax
from jax.experimental import pallas as pl
from jax.experimental.pallas import tpu as pltpu
from jax.experimental.pallas import tpu_sc as plsc

_NC = 2
_NS = 16
_NW = _NC * _NS
_L = 16

_BATCH = 16384
_D = 32
_CARDS = 1000
_W1_ROWS = 1024000

_FB_STRIDE = (_W1_ROWS // 128) * 1024
_CB_STRIDE = 1024
_FI_STRIDE = 128

_mesh = plsc.VectorSubcoreMesh(core_axis_name="c", subcore_axis_name="s")


@functools.partial(
    pl.kernel,
    mesh=_mesh,
    compiler_params=pltpu.CompilerParams(
        use_tc_tiling_on_sc=False, needs_layout_passes=False
    ),
    out_type=(
        jax.ShapeDtypeStruct((_D, _BATCH), jnp.float32),
        jax.ShapeDtypeStruct((_D, _BATCH), jnp.float32),
    ),
    scratch_types=[
        pltpu.VMEM((_BATCH,), jnp.int32),
        pltpu.VMEM((_BATCH,), jnp.int32),
        pltpu.VMEM((_BATCH,), jnp.int32),
        pltpu.VMEM((_BATCH,), jnp.float32),
        pltpu.VMEM((_BATCH,), jnp.float32),
        pltpu.VMEM((1024,), jnp.float32),
        pltpu.SemaphoreType.DMA,
        pltpu.SemaphoreType.DMA,
    ],
)
def _qnet_gather(c0_hbm, c1_hbm, u0_hbm, w0cm_hbm, w1f_hbm, q0t_hbm, q1t_hbm,
                 cA, cB, idx1, col0, col1, w0col, sem0, sem1):
    i32 = jnp.int32
    f = lax.axis_index("s") * i32(_NC) + lax.axis_index("c")
    base1 = (f // i32(8)) * i32(_FB_STRIDE) + (f % i32(8)) * i32(_FI_STRIDE)

    pltpu.sync_copy(c1_hbm, cA)
    pltpu.sync_copy(u0_hbm, cB)

    @plsc.parallel_loop(i32(0), i32(_BATCH), i32(_L), unroll=8)
    def _mk_idx(i):
        sl = pl.ds(i, _L)
        t = cB[sl] * i32(33)
        idx1[sl] = base1 + cA[sl] * i32(8192) + (t >> i32(7)) * i32(1024) + (t & i32(127))

    g1 = pltpu.async_copy(w1f_hbm.at[idx1], col1, sem1)

    pltpu.sync_copy(c0_hbm, cA)
    pltpu.sync_copy(w0cm_hbm.at[f], w0col)

    @plsc.parallel_loop(i32(0), i32(_BATCH), i32(_L), unroll=8)
    def _q0(i):
        sl = pl.ds(i, _L)
        col0[sl] = plsc.load_gather(w0col, [cA[sl]])

    out0 = pltpu.async_copy(col0, q0t_hbm.at[f], sem0)
    g1.wait()
    out1 = pltpu.async_copy(col1, q1t_hbm.at[f], sem1)
    out0.wait()
    out1.wait()


def kernel(cards_0, cards_1, u0, weights_0, weights_1):
    c0 = cards_0.astype(jnp.int32)
    c1 = cards_1.astype(jnp.int32)
    u0i = u0.astype(jnp.int32)
    w1f = (
        weights_1.T.reshape(4, 8, _W1_ROWS // 128, 128)
        .transpose(0, 2, 1, 3)
        .reshape(-1)
    )
    w0cm = jnp.pad(weights_0, ((0, 1024 - _CARDS), (0, 0))).T
    q0t, q1t = _qnet_gather(c0, c1, u0i, w0cm, w1f)
    return (q0t.T, q1t.T)


# speedup vs baseline: 8.5092x; 1.0107x over previous
import functools

import jax
import jax.numpy as jnp
from jax import lax
from jax.experimental import pallas as pl
from jax.experimental.pallas import tpu as pltpu
from jax.experimental.pallas import tpu_sc as plsc

_NC = 2
_NS = 16
_NW = _NC * _NS
_L = 16

_BATCH = 16384
_D = 32
_CARDS = 1000
_W1_ROWS = 1024000

_FB_STRIDE = (_W1_ROWS // 128) * 1024
_CB_STRIDE = 1024
_FI_STRIDE = 128

_mesh = plsc.VectorSubcoreMesh(core_axis_name="c", subcore_axis_name="s")


@functools.partial(
    pl.kernel,
    mesh=_mesh,
    compiler_params=pltpu.CompilerParams(
        use_tc_tiling_on_sc=False,
        needs_layout_passes=False,
        disable_bounds_checks=True,
    ),
    out_type=(
        jax.ShapeDtypeStruct((_D, _BATCH), jnp.float32),
        jax.ShapeDtypeStruct((_D, _BATCH), jnp.float32),
    ),
    scratch_types=[
        pltpu.VMEM((_BATCH,), jnp.int32),
        pltpu.VMEM((_BATCH,), jnp.int32),
        pltpu.VMEM((_BATCH,), jnp.int32),
        pltpu.VMEM((_BATCH,), jnp.float32),
        pltpu.VMEM((_BATCH,), jnp.float32),
        pltpu.VMEM((1024,), jnp.float32),
        pltpu.SemaphoreType.DMA,
        pltpu.SemaphoreType.DMA,
    ],
)
def _qnet_gather(c0_hbm, c1_hbm, u0_hbm, w0cm_hbm, w1f_hbm, q0t_hbm, q1t_hbm,
                 cA, cB, idx1, col0, col1, w0col, sem0, sem1):
    i32 = jnp.int32
    f = lax.axis_index("s") * i32(_NC) + lax.axis_index("c")
    base1 = (f // i32(8)) * i32(_FB_STRIDE) + (f % i32(8)) * i32(_FI_STRIDE)

    pltpu.sync_copy(c1_hbm, cA)
    pltpu.sync_copy(u0_hbm, cB)

    seg = _BATCH // 8
    gathers = []
    for s in range(8):
        @plsc.parallel_loop(i32(s * seg), i32((s + 1) * seg), i32(_L), unroll=8)
        def _mk_idx(i):
            sl = pl.ds(i, _L)
            t = cB[sl] * i32(33)
            idx1[sl] = base1 + cA[sl] * i32(8192) + (t >> i32(7)) * i32(1024) + (t & i32(127))

        gathers.append(
            pltpu.async_copy(
                w1f_hbm.at[idx1.at[pl.ds(s * seg, seg)]],
                col1.at[pl.ds(s * seg, seg)],
                sem1,
            )
        )

    pltpu.sync_copy(c0_hbm, cA)
    pltpu.sync_copy(w0cm_hbm.at[f], w0col)

    @plsc.parallel_loop(i32(0), i32(_BATCH), i32(_L), unroll=8)
    def _q0(i):
        sl = pl.ds(i, _L)
        col0[sl] = plsc.load_gather(w0col, [cA[sl]])

    out0 = pltpu.async_copy(col0, q0t_hbm.at[f], sem0)
    for g in gathers:
        g.wait()
    out1 = pltpu.async_copy(col1, q1t_hbm.at[f], sem1)
    out0.wait()
    out1.wait()


def kernel(cards_0, cards_1, u0, weights_0, weights_1):
    c0 = cards_0.astype(jnp.int32)
    c1 = cards_1.astype(jnp.int32)
    u0i = u0.astype(jnp.int32)
    w1f = (
        weights_1.T.reshape(4, 8, _W1_ROWS // 128, 128)
        .transpose(0, 2, 1, 3)
        .reshape(-1)
    )
    w0cm = jnp.pad(weights_0, ((0, 1024 - _CARDS), (0, 0))).T
    q0t, q1t = _qnet_gather(c0, c1, u0i, w0cm, w1f)
    return (q0t.T, q1t.T)
